# baseline (speedup 1.0000x reference)
import functools

import jax
import jax.numpy as jnp
from jax import lax
from jax.experimental import pallas as pl
from jax.experimental.pallas import tpu as pltpu
from jax.experimental.pallas import tpu_sc as plsc

B = 128
N = 32768
K = 64
L = 16
NCHUNK = N // L
GROUP = 16
NGROUP = NCHUNK // GROUP
WARM_CHUNKS = 256
NWORKERS = 32
ROWS_PER_W = B // NWORKERS
TOL = 1e-10


def _hit(v, thr):
    return plsc.all_reduce_population_count(v > thr)[0] > 0


def _sort_desc(v):
    k, _ = plsc.sort_key_val(v, v, descending=True)
    return k


def _sort_asc(v):
    k, _ = plsc.sort_key_val(v, v, descending=False)
    return k


def _rev(v):
    return lax.rev(v, (0,))


def _fold64(b0, b1, b2, b3, c0, c1, c2, c3):
    s0, s1 = _sort_asc(c0), _sort_asc(c1)
    s2, s3 = _sort_asc(c2), _sort_asc(c3)
    h0 = _sort_desc(jnp.maximum(s0, _rev(s1)))
    l0 = _sort_desc(jnp.minimum(s0, _rev(s1)))
    h1 = _sort_desc(jnp.maximum(s2, _rev(s3)))
    l1 = _sort_desc(jnp.minimum(s2, _rev(s3)))
    rb0, rb1 = _rev(l1), _rev(h1)
    t0, t1 = jnp.maximum(h0, rb0), jnp.maximum(l0, rb1)
    u0, u1 = jnp.minimum(h0, rb0), jnp.minimum(l0, rb1)
    g0 = _sort_desc(jnp.maximum(t0, t1))
    g1 = _sort_desc(jnp.minimum(t0, t1))
    g2 = _sort_desc(jnp.maximum(u0, u1))
    g3 = _sort_desc(jnp.minimum(u0, u1))
    m0 = jnp.maximum(b0, _rev(g3))
    m1 = jnp.maximum(b1, _rev(g2))
    m2 = jnp.maximum(b2, _rev(g1))
    m3 = jnp.maximum(b3, _rev(g0))
    a0, a2 = jnp.maximum(m0, m2), jnp.minimum(m0, m2)
    a1, a3 = jnp.maximum(m1, m3), jnp.minimum(m1, m3)
    n0 = _sort_desc(jnp.maximum(a0, a1))
    n1 = _sort_desc(jnp.minimum(a0, a1))
    n2 = _sort_desc(jnp.maximum(a2, a3))
    n3 = _sort_desc(jnp.minimum(a2, a3))
    return n0, n1, n2, n3


def _merge_refs(ca, cmax, top_v, thr_s):
    t48 = thr_s[1]
    t32 = thr_s[2]
    b3 = top_v[pl.ds(3 * L, L)]
    t = jnp.maximum(b3, ca)

    @pl.when(cmax < t48)
    def _():
        n3 = _sort_desc(t)
        top_v[pl.ds(3 * L, L)] = n3
        thr_s[0] = n3[L - 1]

    @pl.when((cmax >= t48) & (cmax < t32))
    def _():
        b2 = top_v[pl.ds(2 * L, L)]
        n2 = _sort_desc(jnp.maximum(b2, t))
        n3 = _sort_desc(jnp.minimum(b2, t))
        top_v[pl.ds(2 * L, L)] = n2
        top_v[pl.ds(3 * L, L)] = n3
        thr_s[0] = n3[L - 1]
        thr_s[1] = n2[L - 1]

    @pl.when(cmax >= t32)
    def _():
        b0 = top_v[pl.ds(0, L)]
        b1 = top_v[pl.ds(L, L)]
        b2 = top_v[pl.ds(2 * L, L)]
        a1 = jnp.maximum(b1, t)
        a3 = jnp.minimum(b1, t)
        n0 = _sort_desc(jnp.maximum(b0, a1))
        n1 = _sort_desc(jnp.minimum(b0, a1))
        n2 = _sort_desc(jnp.maximum(b2, a3))
        n3 = _sort_desc(jnp.minimum(b2, a3))
        top_v[pl.ds(0, L)] = n0
        top_v[pl.ds(L, L)] = n1
        top_v[pl.ds(2 * L, L)] = n2
        top_v[pl.ds(3 * L, L)] = n3
        thr_s[0] = n3[L - 1]
        thr_s[1] = n2[L - 1]
        thr_s[2] = n1[L - 1]


def _sc_body(x_hbm, w_hbm, out_hbm, row_a, row_b, w_v, res_v, top_v, thr_s,
             sem0, sem1):
    wid = lax.axis_index("s") * 2 + lax.axis_index("c")
    base = wid * ROWS_PER_W
    lane = lax.iota(jnp.int32, L)
    res = jnp.zeros((L,), jnp.float32)

    bufs = (row_a, row_b)
    sems = (sem0, sem1)
    pltpu.sync_copy(w_hbm.at[pl.ds(base, ROWS_PER_W)], w_v)
    copies = [None, None]
    copies[0] = pltpu.make_async_copy(x_hbm.at[base], row_a, sem0)
    copies[0].start()

    for j in range(ROWS_PER_W):
        cur = j % 2
        copies[cur].wait()
        if j + 1 < ROWS_PER_W:
            nxt = (j + 1) % 2
            copies[nxt] = pltpu.make_async_copy(
                x_hbm.at[base + j + 1], bufs[nxt], sems[nxt])
            copies[nxt].start()
        row_v = bufs[cur]

        neg = jnp.float32(-jnp.inf)

        def warm_body(i, bs):
            off = i * (4 * L)
            return _fold64(*bs,
                           row_v[pl.ds(off, L)],
                           row_v[pl.ds(off + L, L)],
                           row_v[pl.ds(off + 2 * L, L)],
                           row_v[pl.ds(off + 3 * L, L)])

        w0, w1, w2, w3 = lax.fori_loop(
            0, WARM_CHUNKS // 4, warm_body,
            (jnp.full((L,), neg), jnp.full((L,), neg),
             jnp.full((L,), neg), jnp.full((L,), neg)))
        top_v[pl.ds(0, L)] = w0
        top_v[pl.ds(L, L)] = w1
        top_v[pl.ds(2 * L, L)] = w2
        top_v[pl.ds(3 * L, L)] = w3
        thr_s[0] = w3[L - 1]
        thr_s[1] = w2[L - 1]
        thr_s[2] = w1[L - 1]

        def group_body(g, dummy):
            off = g * (GROUP * L)
            cs = [row_v[pl.ds(off + i * L, L)] for i in range(GROUP)]
            pairs = [jnp.maximum(cs[i], cs[i + 1])
                     for i in range(0, GROUP, 2)]
            quads = [jnp.maximum(pairs[i], pairs[i + 1])
                     for i in range(0, len(pairs), 2)]
            gm = quads
            while len(gm) > 1:
                gm = [jnp.maximum(gm[i], gm[i + 1])
                      for i in range(0, len(gm), 2)]

            @pl.when(_hit(gm[0], thr_s[0]))
            def _():
                for qi, q in enumerate(quads):
                    @pl.when(_hit(q, thr_s[0]))
                    def _(qi=qi):
                        cas = [_sort_asc(cs[4 * qi + i]) for i in range(4)]
                        for ca in cas:
                            cmax = ca[L - 1]

                            @pl.when(cmax > thr_s[0])
                            def _(ca=ca, cmax=cmax):
                                _merge_refs(ca, cmax, top_v, thr_s)
            return dummy

        lax.fori_loop(WARM_CHUNKS // GROUP, NGROUP, group_body, 0)
        b0 = top_v[pl.ds(0, L)]
        b1 = top_v[pl.ds(L, L)]
        b2 = top_v[pl.ds(2 * L, L)]
        b3 = top_v[pl.ds(3 * L, L)]

        acc = b0 * w_v[j, pl.ds(0, L)]
        acc = acc + b1 * w_v[j, pl.ds(L, L)]
        acc = acc + b2 * w_v[j, pl.ds(2 * L, L)]
        acc = acc + b3 * w_v[j, pl.ds(3 * L, L)]
        s = jnp.sum(acc)
        res = jnp.where(lane == j, s, res)

    res_v[...] = res
    pltpu.sync_copy(res_v, out_hbm.at[wid])


_topk_dot_sc = functools.partial(
    pl.kernel,
    out_type=jax.ShapeDtypeStruct((NWORKERS, L), jnp.float32),
    mesh=plsc.VectorSubcoreMesh(core_axis_name="c", subcore_axis_name="s"),
    scratch_types=[
        pltpu.VMEM((N,), jnp.float32),
        pltpu.VMEM((N,), jnp.float32),
        pltpu.VMEM((ROWS_PER_W, K), jnp.float32),
        pltpu.VMEM((L,), jnp.float32),
        pltpu.VMEM((K,), jnp.float32),
        pltpu.SMEM((4,), jnp.float32),
        pltpu.SemaphoreType.DMA,
        pltpu.SemaphoreType.DMA,
    ],
    compiler_params=pltpu.CompilerParams(needs_layout_passes=False),
)(_sc_body)


def _logmean_body(x_ref, o_ref):
    x = x_ref[...]
    lane_id = lax.broadcasted_iota(jnp.int32, (NWORKERS, L), 1)
    v = jnp.where(lane_id < ROWS_PER_W, jnp.log(x + TOL), 0.0)
    o_ref[...] = (jnp.sum(v) * (1.0 / B))[None, None]


def kernel(concat_output, sample_loc_prob):
    staged = _topk_dot_sc(concat_output, sample_loc_prob)
    out = pl.pallas_call(
        _logmean_body,
        out_shape=jax.ShapeDtypeStruct((1, 1), jnp.float32),
    )(staged)
    return out[0, 0]


# speedup vs baseline: 1.2095x; 1.2095x over previous
import functools

import jax
import jax.numpy as jnp
from jax import lax
from jax.experimental import pallas as pl
from jax.experimental.pallas import tpu as pltpu
from jax.experimental.pallas import tpu_sc as plsc

B = 128
N = 32768
K = 64
L = 16
NCHUNK = N // L
GROUP = 16
NGROUP = NCHUNK // GROUP
WARM_CHUNKS = 384
NWORKERS = 32
ROWS_PER_W = B // NWORKERS
TOL = 1e-10


def _hit(v, thr):
    return plsc.all_reduce_population_count(v > thr)[0] > 0


def _sort_desc(v):
    k, _ = plsc.sort_key_val(v, v, descending=True)
    return k


def _sort_asc(v):
    k, _ = plsc.sort_key_val(v, v, descending=False)
    return k


def _rev(v):
    return lax.rev(v, (0,))


def _fold64(b0, b1, b2, b3, c0, c1, c2, c3):
    s0, s1 = _sort_asc(c0), _sort_asc(c1)
    s2, s3 = _sort_asc(c2), _sort_asc(c3)
    h0 = _sort_desc(jnp.maximum(s0, _rev(s1)))
    l0 = _sort_desc(jnp.minimum(s0, _rev(s1)))
    h1 = _sort_desc(jnp.maximum(s2, _rev(s3)))
    l1 = _sort_desc(jnp.minimum(s2, _rev(s3)))
    rb0, rb1 = _rev(l1), _rev(h1)
    t0, t1 = jnp.maximum(h0, rb0), jnp.maximum(l0, rb1)
    u0, u1 = jnp.minimum(h0, rb0), jnp.minimum(l0, rb1)
    g0 = _sort_desc(jnp.maximum(t0, t1))
    g1 = _sort_desc(jnp.minimum(t0, t1))
    g2 = _sort_desc(jnp.maximum(u0, u1))
    g3 = _sort_desc(jnp.minimum(u0, u1))
    m0 = jnp.maximum(b0, _rev(g3))
    m1 = jnp.maximum(b1, _rev(g2))
    m2 = jnp.maximum(b2, _rev(g1))
    m3 = jnp.maximum(b3, _rev(g0))
    a0, a2 = jnp.maximum(m0, m2), jnp.minimum(m0, m2)
    a1, a3 = jnp.maximum(m1, m3), jnp.minimum(m1, m3)
    n0 = _sort_desc(jnp.maximum(a0, a1))
    n1 = _sort_desc(jnp.minimum(a0, a1))
    n2 = _sort_desc(jnp.maximum(a2, a3))
    n3 = _sort_desc(jnp.minimum(a2, a3))
    return n0, n1, n2, n3


def _merge(c, b0, b1, b2, b3):
    ca = _sort_asc(c)
    cmax = ca[L - 1]
    return _merge_sorted(ca, cmax, b0, b1, b2, b3)


def _merge_sorted(ca, cmax, b0, b1, b2, b3):
    def b3_only():
        n3 = _sort_desc(jnp.maximum(b3, ca))
        return b0, b1, b2, n3, n3[L - 1]

    def b23():
        t = jnp.maximum(b3, ca)
        n2 = _sort_desc(jnp.maximum(b2, t))
        n3 = _sort_desc(jnp.minimum(b2, t))
        return b0, b1, n2, n3, n3[L - 1]

    def full():
        t = jnp.maximum(b3, ca)
        a1 = jnp.maximum(b1, t)
        a3 = jnp.minimum(b1, t)
        n0 = _sort_desc(jnp.maximum(b0, a1))
        n1 = _sort_desc(jnp.minimum(b0, a1))
        n2 = _sort_desc(jnp.maximum(b2, a3))
        n3 = _sort_desc(jnp.minimum(b2, a3))
        return n0, n1, n2, n3, n3[L - 1]

    return lax.cond(
        cmax < b2[L - 1], b3_only,
        lambda: lax.cond(cmax < b1[L - 1], b23, full))


def _sc_body(x_hbm, w_hbm, out_hbm, row_a, row_b, w_v, res_v, sem0, sem1):
    wid = lax.axis_index("s") * 2 + lax.axis_index("c")
    base = wid * ROWS_PER_W
    lane = lax.iota(jnp.int32, L)
    res = jnp.zeros((L,), jnp.float32)

    bufs = (row_a, row_b)
    sems = (sem0, sem1)
    pltpu.sync_copy(w_hbm.at[pl.ds(base, ROWS_PER_W)], w_v)
    copies = [None, None]
    copies[0] = pltpu.make_async_copy(x_hbm.at[base], row_a, sem0)
    copies[0].start()

    for j in range(ROWS_PER_W):
        cur = j % 2
        copies[cur].wait()
        if j + 1 < ROWS_PER_W:
            nxt = (j + 1) % 2
            copies[nxt] = pltpu.make_async_copy(
                x_hbm.at[base + j + 1], bufs[nxt], sems[nxt])
            copies[nxt].start()
        row_v = bufs[cur]

        neg = jnp.float32(-jnp.inf)

        def warm_body(i, bs):
            off = i * (4 * L)
            return _fold64(*bs,
                           row_v[pl.ds(off, L)],
                           row_v[pl.ds(off + L, L)],
                           row_v[pl.ds(off + 2 * L, L)],
                           row_v[pl.ds(off + 3 * L, L)])

        w0, w1, w2, w3 = lax.fori_loop(
            0, WARM_CHUNKS // 4, warm_body,
            (jnp.full((L,), neg), jnp.full((L,), neg),
             jnp.full((L,), neg), jnp.full((L,), neg)))
        init = (w0, w1, w2, w3, w3[L - 1])

        def group_body(g, carry):
            b0, b1, b2, b3, thr = carry
            off = g * (GROUP * L)
            cs = [row_v[pl.ds(off + i * L, L)] for i in range(GROUP)]
            pairs = [jnp.maximum(cs[i], cs[i + 1])
                     for i in range(0, GROUP, 2)]
            quads = [jnp.maximum(pairs[i], pairs[i + 1])
                     for i in range(0, len(pairs), 2)]
            gm = quads
            while len(gm) > 1:
                gm = [jnp.maximum(gm[i], gm[i + 1])
                      for i in range(0, len(gm), 2)]

            def do(carry):
                for qi, q in enumerate(quads):
                    def do_quad(carry, qi=qi):
                        cas = [_sort_asc(cs[4 * qi + i]) for i in range(4)]
                        for ca in cas:
                            cmax = ca[L - 1]

                            def m(cr, ca=ca, cmax=cmax):
                                return _merge_sorted(ca, cmax, cr[0], cr[1],
                                                     cr[2], cr[3])

                            carry = lax.cond(cmax > carry[4], m,
                                             lambda cr: cr, carry)
                        return carry

                    carry = lax.cond(_hit(q, carry[4]), do_quad,
                                     lambda cr: cr, carry)
                return carry

            return lax.cond(_hit(gm[0], thr), do, lambda cr: cr, carry)

        b0, b1, b2, b3, _ = lax.fori_loop(WARM_CHUNKS // GROUP, NGROUP,
                                          group_body, init)

        acc = b0 * w_v[j, pl.ds(0, L)]
        acc = acc + b1 * w_v[j, pl.ds(L, L)]
        acc = acc + b2 * w_v[j, pl.ds(2 * L, L)]
        acc = acc + b3 * w_v[j, pl.ds(3 * L, L)]
        s = jnp.sum(acc)
        res = jnp.where(lane == j, s, res)

    res_v[...] = res
    pltpu.sync_copy(res_v, out_hbm.at[wid])


_topk_dot_sc = functools.partial(
    pl.kernel,
    out_type=jax.ShapeDtypeStruct((NWORKERS, L), jnp.float32),
    mesh=plsc.VectorSubcoreMesh(core_axis_name="c", subcore_axis_name="s"),
    scratch_types=[
        pltpu.VMEM((N,), jnp.float32),
        pltpu.VMEM((N,), jnp.float32),
        pltpu.VMEM((ROWS_PER_W, K), jnp.float32),
        pltpu.VMEM((L,), jnp.float32),
        pltpu.SemaphoreType.DMA,
        pltpu.SemaphoreType.DMA,
    ],
    compiler_params=pltpu.CompilerParams(needs_layout_passes=False),
)(_sc_body)


def _logmean_body(x_ref, o_ref):
    x = x_ref[...]
    lane_id = lax.broadcasted_iota(jnp.int32, (NWORKERS, L), 1)
    v = jnp.where(lane_id < ROWS_PER_W, jnp.log(x + TOL), 0.0)
    o_ref[...] = (jnp.sum(v) * (1.0 / B))[None, None]


def kernel(concat_output, sample_loc_prob):
    staged = _topk_dot_sc(concat_output, sample_loc_prob)
    out = pl.pallas_call(
        _logmean_body,
        out_shape=jax.ShapeDtypeStruct((1, 1), jnp.float32),
    )(staged)
    return out[0, 0]


# speedup vs baseline: 1.3044x; 1.0785x over previous
import functools

import jax
import jax.numpy as jnp
from jax import lax
from jax.experimental import pallas as pl
from jax.experimental.pallas import tpu as pltpu
from jax.experimental.pallas import tpu_sc as plsc

B = 128
N = 32768
K = 64
L = 16
NCHUNK = N // L
GROUP = 16
NGROUP = NCHUNK // GROUP
WARM_CHUNKS = 512
NWORKERS = 32
ROWS_PER_W = B // NWORKERS
TOL = 1e-10


def _hit(v, thr):
    return plsc.all_reduce_population_count(v > thr)[0] > 0


def _sort_desc(v):
    k, _ = plsc.sort_key_val(v, v, descending=True)
    return k


def _sort_asc(v):
    k, _ = plsc.sort_key_val(v, v, descending=False)
    return k


def _rev(v):
    return lax.rev(v, (0,))


def _fold64(b0, b1, b2, b3, c0, c1, c2, c3):
    s0, s1 = _sort_asc(c0), _sort_asc(c1)
    s2, s3 = _sort_asc(c2), _sort_asc(c3)
    h0 = _sort_desc(jnp.maximum(s0, _rev(s1)))
    l0 = _sort_desc(jnp.minimum(s0, _rev(s1)))
    h1 = _sort_desc(jnp.maximum(s2, _rev(s3)))
    l1 = _sort_desc(jnp.minimum(s2, _rev(s3)))
    rb0, rb1 = _rev(l1), _rev(h1)
    t0, t1 = jnp.maximum(h0, rb0), jnp.maximum(l0, rb1)
    u0, u1 = jnp.minimum(h0, rb0), jnp.minimum(l0, rb1)
    g0 = _sort_desc(jnp.maximum(t0, t1))
    g1 = _sort_desc(jnp.minimum(t0, t1))
    g2 = _sort_desc(jnp.maximum(u0, u1))
    g3 = _sort_desc(jnp.minimum(u0, u1))
    m0 = jnp.maximum(b0, _rev(g3))
    m1 = jnp.maximum(b1, _rev(g2))
    m2 = jnp.maximum(b2, _rev(g1))
    m3 = jnp.maximum(b3, _rev(g0))
    a0, a2 = jnp.maximum(m0, m2), jnp.minimum(m0, m2)
    a1, a3 = jnp.maximum(m1, m3), jnp.minimum(m1, m3)
    n0 = _sort_desc(jnp.maximum(a0, a1))
    n1 = _sort_desc(jnp.minimum(a0, a1))
    n2 = _sort_desc(jnp.maximum(a2, a3))
    n3 = _sort_desc(jnp.minimum(a2, a3))
    return n0, n1, n2, n3


def _merge(c, b0, b1, b2, b3):
    ca = _sort_asc(c)
    cmax = ca[L - 1]
    return _merge_sorted(ca, cmax, b0, b1, b2, b3)


def _merge_sorted(ca, cmax, b0, b1, b2, b3):
    def b3_only():
        n3 = _sort_desc(jnp.maximum(b3, ca))
        return b0, b1, b2, n3, n3[L - 1]

    def b23():
        t = jnp.maximum(b3, ca)
        n2 = _sort_desc(jnp.maximum(b2, t))
        n3 = _sort_desc(jnp.minimum(b2, t))
        return b0, b1, n2, n3, n3[L - 1]

    def full():
        t = jnp.maximum(b3, ca)
        a1 = jnp.maximum(b1, t)
        a3 = jnp.minimum(b1, t)
        n0 = _sort_desc(jnp.maximum(b0, a1))
        n1 = _sort_desc(jnp.minimum(b0, a1))
        n2 = _sort_desc(jnp.maximum(b2, a3))
        n3 = _sort_desc(jnp.minimum(b2, a3))
        return n0, n1, n2, n3, n3[L - 1]

    return lax.cond(
        cmax < b2[L - 1], b3_only,
        lambda: lax.cond(cmax < b1[L - 1], b23, full))


def _sc_body(x_hbm, w_hbm, out_hbm, row_a, row_b, w_v, res_v, sem0, sem1):
    wid = lax.axis_index("s") * 2 + lax.axis_index("c")
    base = wid * ROWS_PER_W
    lane = lax.iota(jnp.int32, L)
    res = jnp.zeros((L,), jnp.float32)

    bufs = (row_a, row_b)
    sems = (sem0, sem1)
    pltpu.sync_copy(w_hbm.at[pl.ds(base, ROWS_PER_W)], w_v)
    copies = [None, None]
    copies[0] = pltpu.make_async_copy(x_hbm.at[base], row_a, sem0)
    copies[0].start()

    for j in range(ROWS_PER_W):
        cur = j % 2
        copies[cur].wait()
        if j + 1 < ROWS_PER_W:
            nxt = (j + 1) % 2
            copies[nxt] = pltpu.make_async_copy(
                x_hbm.at[base + j + 1], bufs[nxt], sems[nxt])
            copies[nxt].start()
        row_v = bufs[cur]

        neg = jnp.float32(-jnp.inf)

        def warm_body(i, bs):
            off = i * (4 * L)
            return _fold64(*bs,
                           row_v[pl.ds(off, L)],
                           row_v[pl.ds(off + L, L)],
                           row_v[pl.ds(off + 2 * L, L)],
                           row_v[pl.ds(off + 3 * L, L)])

        w0, w1, w2, w3 = lax.fori_loop(
            0, WARM_CHUNKS // 4, warm_body,
            (jnp.full((L,), neg), jnp.full((L,), neg),
             jnp.full((L,), neg), jnp.full((L,), neg)))
        init = (w0, w1, w2, w3, w3[L - 1])

        def group_body(g, carry):
            b0, b1, b2, b3, thr = carry
            off = g * (GROUP * L)
            cs = [row_v[pl.ds(off + i * L, L)] for i in range(GROUP)]
            pairs = [jnp.maximum(cs[i], cs[i + 1])
                     for i in range(0, GROUP, 2)]
            quads = [jnp.maximum(pairs[i], pairs[i + 1])
                     for i in range(0, len(pairs), 2)]
            gm = quads
            while len(gm) > 1:
                gm = [jnp.maximum(gm[i], gm[i + 1])
                      for i in range(0, len(gm), 2)]

            def do(carry):
                for qi, q in enumerate(quads):
                    def do_quad(carry, qi=qi):
                        cas = [_sort_asc(cs[4 * qi + i]) for i in range(4)]
                        for ca in cas:
                            cmax = ca[L - 1]

                            def m(cr, ca=ca, cmax=cmax):
                                return _merge_sorted(ca, cmax, cr[0], cr[1],
                                                     cr[2], cr[3])

                            carry = lax.cond(cmax > carry[4], m,
                                             lambda cr: cr, carry)
                        return carry

                    carry = lax.cond(_hit(q, carry[4]), do_quad,
                                     lambda cr: cr, carry)
                return carry

            return lax.cond(_hit(gm[0], thr), do, lambda cr: cr, carry)

        b0, b1, b2, b3, _ = lax.fori_loop(WARM_CHUNKS // GROUP, NGROUP,
                                          group_body, init)

        acc = b0 * w_v[j, pl.ds(0, L)]
        acc = acc + b1 * w_v[j, pl.ds(L, L)]
        acc = acc + b2 * w_v[j, pl.ds(2 * L, L)]
        acc = acc + b3 * w_v[j, pl.ds(3 * L, L)]
        s = jnp.sum(acc)
        res = jnp.where(lane == j, s, res)

    res_v[...] = res
    pltpu.sync_copy(res_v, out_hbm.at[wid])


_topk_dot_sc = functools.partial(
    pl.kernel,
    out_type=jax.ShapeDtypeStruct((NWORKERS, L), jnp.float32),
    mesh=plsc.VectorSubcoreMesh(core_axis_name="c", subcore_axis_name="s"),
    scratch_types=[
        pltpu.VMEM((N,), jnp.float32),
        pltpu.VMEM((N,), jnp.float32),
        pltpu.VMEM((ROWS_PER_W, K), jnp.float32),
        pltpu.VMEM((L,), jnp.float32),
        pltpu.SemaphoreType.DMA,
        pltpu.SemaphoreType.DMA,
    ],
    compiler_params=pltpu.CompilerParams(needs_layout_passes=False),
)(_sc_body)


def _logmean_body(x_ref, o_ref):
    x = x_ref[...]
    lane_id = lax.broadcasted_iota(jnp.int32, (NWORKERS, L), 1)
    v = jnp.where(lane_id < ROWS_PER_W, jnp.log(x + TOL), 0.0)
    o_ref[...] = (jnp.sum(v) * (1.0 / B))[None, None]


def kernel(concat_output, sample_loc_prob):
    staged = _topk_dot_sc(concat_output, sample_loc_prob)
    out = pl.pallas_call(
        _logmean_body,
        out_shape=jax.ShapeDtypeStruct((1, 1), jnp.float32),
    )(staged)
    return out[0, 0]


# speedup vs baseline: 1.4722x; 1.1286x over previous
import functools

import jax
import jax.numpy as jnp
from jax import lax
from jax.experimental import pallas as pl
from jax.experimental.pallas import tpu as pltpu
from jax.experimental.pallas import tpu_sc as plsc

B = 128
N = 32768
K = 64
L = 16
NCHUNK = N // L
GROUP = 16
NGROUP = NCHUNK // GROUP
WARM_CHUNKS = 768
NWORKERS = 32
ROWS_PER_W = B // NWORKERS
TOL = 1e-10


def _hit(v, thr):
    return plsc.all_reduce_population_count(v > thr)[0] > 0


def _sort_desc(v):
    k, _ = plsc.sort_key_val(v, v, descending=True)
    return k


def _sort_asc(v):
    k, _ = plsc.sort_key_val(v, v, descending=False)
    return k


def _rev(v):
    return lax.rev(v, (0,))


def _fold64(b0, b1, b2, b3, c0, c1, c2, c3):
    s0, s1 = _sort_asc(c0), _sort_asc(c1)
    s2, s3 = _sort_asc(c2), _sort_asc(c3)
    h0 = _sort_desc(jnp.maximum(s0, _rev(s1)))
    l0 = _sort_desc(jnp.minimum(s0, _rev(s1)))
    h1 = _sort_desc(jnp.maximum(s2, _rev(s3)))
    l1 = _sort_desc(jnp.minimum(s2, _rev(s3)))
    rb0, rb1 = _rev(l1), _rev(h1)
    t0, t1 = jnp.maximum(h0, rb0), jnp.maximum(l0, rb1)
    u0, u1 = jnp.minimum(h0, rb0), jnp.minimum(l0, rb1)
    g0 = _sort_desc(jnp.maximum(t0, t1))
    g1 = _sort_desc(jnp.minimum(t0, t1))
    g2 = _sort_desc(jnp.maximum(u0, u1))
    g3 = _sort_desc(jnp.minimum(u0, u1))
    m0 = jnp.maximum(b0, _rev(g3))
    m1 = jnp.maximum(b1, _rev(g2))
    m2 = jnp.maximum(b2, _rev(g1))
    m3 = jnp.maximum(b3, _rev(g0))
    a0, a2 = jnp.maximum(m0, m2), jnp.minimum(m0, m2)
    a1, a3 = jnp.maximum(m1, m3), jnp.minimum(m1, m3)
    n0 = _sort_desc(jnp.maximum(a0, a1))
    n1 = _sort_desc(jnp.minimum(a0, a1))
    n2 = _sort_desc(jnp.maximum(a2, a3))
    n3 = _sort_desc(jnp.minimum(a2, a3))
    return n0, n1, n2, n3


def _merge(c, b0, b1, b2, b3):
    ca = _sort_asc(c)
    cmax = ca[L - 1]
    return _merge_sorted(ca, cmax, b0, b1, b2, b3)


def _merge_sorted(ca, cmax, b0, b1, b2, b3):
    def b3_only():
        n3 = _sort_desc(jnp.maximum(b3, ca))
        return b0, b1, b2, n3, n3[L - 1]

    def b23():
        t = jnp.maximum(b3, ca)
        n2 = _sort_desc(jnp.maximum(b2, t))
        n3 = _sort_desc(jnp.minimum(b2, t))
        return b0, b1, n2, n3, n3[L - 1]

    def full():
        t = jnp.maximum(b3, ca)
        a1 = jnp.maximum(b1, t)
        a3 = jnp.minimum(b1, t)
        n0 = _sort_desc(jnp.maximum(b0, a1))
        n1 = _sort_desc(jnp.minimum(b0, a1))
        n2 = _sort_desc(jnp.maximum(b2, a3))
        n3 = _sort_desc(jnp.minimum(b2, a3))
        return n0, n1, n2, n3, n3[L - 1]

    return lax.cond(
        cmax < b2[L - 1], b3_only,
        lambda: lax.cond(cmax < b1[L - 1], b23, full))


def _sc_body(x_hbm, w_hbm, out_hbm, row_a, row_b, w_v, res_v, sem0, sem1):
    wid = lax.axis_index("s") * 2 + lax.axis_index("c")
    base = wid * ROWS_PER_W
    lane = lax.iota(jnp.int32, L)
    res = jnp.zeros((L,), jnp.float32)

    bufs = (row_a, row_b)
    sems = (sem0, sem1)
    pltpu.sync_copy(w_hbm.at[pl.ds(base, ROWS_PER_W)], w_v)
    copies = [None, None]
    copies[0] = pltpu.make_async_copy(x_hbm.at[base], row_a, sem0)
    copies[0].start()

    for j in range(ROWS_PER_W):
        cur = j % 2
        copies[cur].wait()
        if j + 1 < ROWS_PER_W:
            nxt = (j + 1) % 2
            copies[nxt] = pltpu.make_async_copy(
                x_hbm.at[base + j + 1], bufs[nxt], sems[nxt])
            copies[nxt].start()
        row_v = bufs[cur]

        neg = jnp.float32(-jnp.inf)

        def warm_body(i, bs):
            off = i * (4 * L)
            return _fold64(*bs,
                           row_v[pl.ds(off, L)],
                           row_v[pl.ds(off + L, L)],
                           row_v[pl.ds(off + 2 * L, L)],
                           row_v[pl.ds(off + 3 * L, L)])

        w0, w1, w2, w3 = lax.fori_loop(
            0, WARM_CHUNKS // 4, warm_body,
            (jnp.full((L,), neg), jnp.full((L,), neg),
             jnp.full((L,), neg), jnp.full((L,), neg)))
        init = (w0, w1, w2, w3, w3[L - 1])

        def group_body(g, carry):
            b0, b1, b2, b3, thr = carry
            off = g * (GROUP * L)
            cs = [row_v[pl.ds(off + i * L, L)] for i in range(GROUP)]
            pairs = [jnp.maximum(cs[i], cs[i + 1])
                     for i in range(0, GROUP, 2)]
            quads = [jnp.maximum(pairs[i], pairs[i + 1])
                     for i in range(0, len(pairs), 2)]
            gm = quads
            while len(gm) > 1:
                gm = [jnp.maximum(gm[i], gm[i + 1])
                      for i in range(0, len(gm), 2)]

            def do(carry):
                for qi, q in enumerate(quads):
                    def do_quad(carry, qi=qi):
                        cas = [_sort_asc(cs[4 * qi + i]) for i in range(4)]
                        for ca in cas:
                            cmax = ca[L - 1]

                            def m(cr, ca=ca, cmax=cmax):
                                return _merge_sorted(ca, cmax, cr[0], cr[1],
                                                     cr[2], cr[3])

                            carry = lax.cond(cmax > carry[4], m,
                                             lambda cr: cr, carry)
                        return carry

                    carry = lax.cond(_hit(q, carry[4]), do_quad,
                                     lambda cr: cr, carry)
                return carry

            return lax.cond(_hit(gm[0], thr), do, lambda cr: cr, carry)

        b0, b1, b2, b3, _ = lax.fori_loop(WARM_CHUNKS // GROUP, NGROUP,
                                          group_body, init)

        acc = b0 * w_v[j, pl.ds(0, L)]
        acc = acc + b1 * w_v[j, pl.ds(L, L)]
        acc = acc + b2 * w_v[j, pl.ds(2 * L, L)]
        acc = acc + b3 * w_v[j, pl.ds(3 * L, L)]
        s = jnp.sum(acc)
        res = jnp.where(lane == j, s, res)

    res_v[...] = res
    pltpu.sync_copy(res_v, out_hbm.at[wid])


_topk_dot_sc = functools.partial(
    pl.kernel,
    out_type=jax.ShapeDtypeStruct((NWORKERS, L), jnp.float32),
    mesh=plsc.VectorSubcoreMesh(core_axis_name="c", subcore_axis_name="s"),
    scratch_types=[
        pltpu.VMEM((N,), jnp.float32),
        pltpu.VMEM((N,), jnp.float32),
        pltpu.VMEM((ROWS_PER_W, K), jnp.float32),
        pltpu.VMEM((L,), jnp.float32),
        pltpu.SemaphoreType.DMA,
        pltpu.SemaphoreType.DMA,
    ],
    compiler_params=pltpu.CompilerParams(needs_layout_passes=False),
)(_sc_body)


def _logmean_body(x_ref, o_ref):
    x = x_ref[...]
    lane_id = lax.broadcasted_iota(jnp.int32, (NWORKERS, L), 1)
    v = jnp.where(lane_id < ROWS_PER_W, jnp.log(x + TOL), 0.0)
    o_ref[...] = (jnp.sum(v) * (1.0 / B))[None, None]


def kernel(concat_output, sample_loc_prob):
    staged = _topk_dot_sc(concat_output, sample_loc_prob)
    out = pl.pallas_call(
        _logmean_body,
        out_shape=jax.ShapeDtypeStruct((1, 1), jnp.float32),
    )(staged)
    return out[0, 0]


# speedup vs baseline: 2.5617x; 1.7401x over previous
import functools

import jax
import jax.numpy as jnp
from jax import lax
from jax.experimental import pallas as pl
from jax.experimental.pallas import tpu as pltpu
from jax.experimental.pallas import tpu_sc as plsc

B = 128
N = 32768
K = 64
L = 16
NCHUNK = N // L
GROUP = 16
NGROUP = NCHUNK // GROUP
WARM_CHUNKS = 2048
NWORKERS = 32
ROWS_PER_W = B // NWORKERS
TOL = 1e-10


def _hit(v, thr):
    return plsc.all_reduce_population_count(v > thr)[0] > 0


def _sort_desc(v):
    k, _ = plsc.sort_key_val(v, v, descending=True)
    return k


def _sort_asc(v):
    k, _ = plsc.sort_key_val(v, v, descending=False)
    return k


def _rev(v):
    return lax.rev(v, (0,))


def _fold64(b0, b1, b2, b3, c0, c1, c2, c3):
    s0, s1 = _sort_asc(c0), _sort_asc(c1)
    s2, s3 = _sort_asc(c2), _sort_asc(c3)
    h0 = _sort_desc(jnp.maximum(s0, _rev(s1)))
    l0 = _sort_desc(jnp.minimum(s0, _rev(s1)))
    h1 = _sort_desc(jnp.maximum(s2, _rev(s3)))
    l1 = _sort_desc(jnp.minimum(s2, _rev(s3)))
    rb0, rb1 = _rev(l1), _rev(h1)
    t0, t1 = jnp.maximum(h0, rb0), jnp.maximum(l0, rb1)
    u0, u1 = jnp.minimum(h0, rb0), jnp.minimum(l0, rb1)
    g0 = _sort_desc(jnp.maximum(t0, t1))
    g1 = _sort_desc(jnp.minimum(t0, t1))
    g2 = _sort_desc(jnp.maximum(u0, u1))
    g3 = _sort_desc(jnp.minimum(u0, u1))
    m0 = jnp.maximum(b0, _rev(g3))
    m1 = jnp.maximum(b1, _rev(g2))
    m2 = jnp.maximum(b2, _rev(g1))
    m3 = jnp.maximum(b3, _rev(g0))
    a0, a2 = jnp.maximum(m0, m2), jnp.minimum(m0, m2)
    a1, a3 = jnp.maximum(m1, m3), jnp.minimum(m1, m3)
    n0 = _sort_desc(jnp.maximum(a0, a1))
    n1 = _sort_desc(jnp.minimum(a0, a1))
    n2 = _sort_desc(jnp.maximum(a2, a3))
    n3 = _sort_desc(jnp.minimum(a2, a3))
    return n0, n1, n2, n3


def _merge(c, b0, b1, b2, b3):
    ca = _sort_asc(c)
    cmax = ca[L - 1]
    return _merge_sorted(ca, cmax, b0, b1, b2, b3)


def _merge_sorted(ca, cmax, b0, b1, b2, b3):
    def b3_only():
        n3 = _sort_desc(jnp.maximum(b3, ca))
        return b0, b1, b2, n3, n3[L - 1]

    def b23():
        t = jnp.maximum(b3, ca)
        n2 = _sort_desc(jnp.maximum(b2, t))
        n3 = _sort_desc(jnp.minimum(b2, t))
        return b0, b1, n2, n3, n3[L - 1]

    def full():
        t = jnp.maximum(b3, ca)
        a1 = jnp.maximum(b1, t)
        a3 = jnp.minimum(b1, t)
        n0 = _sort_desc(jnp.maximum(b0, a1))
        n1 = _sort_desc(jnp.minimum(b0, a1))
        n2 = _sort_desc(jnp.maximum(b2, a3))
        n3 = _sort_desc(jnp.minimum(b2, a3))
        return n0, n1, n2, n3, n3[L - 1]

    return lax.cond(
        cmax < b2[L - 1], b3_only,
        lambda: lax.cond(cmax < b1[L - 1], b23, full))


def _sc_body(x_hbm, w_hbm, out_hbm, row_a, row_b, w_v, res_v, sem0, sem1):
    wid = lax.axis_index("s") * 2 + lax.axis_index("c")
    base = wid * ROWS_PER_W
    lane = lax.iota(jnp.int32, L)
    res = jnp.zeros((L,), jnp.float32)

    bufs = (row_a, row_b)
    sems = (sem0, sem1)
    pltpu.sync_copy(w_hbm.at[pl.ds(base, ROWS_PER_W)], w_v)
    copies = [None, None]
    copies[0] = pltpu.make_async_copy(x_hbm.at[base], row_a, sem0)
    copies[0].start()

    for j in range(ROWS_PER_W):
        cur = j % 2
        copies[cur].wait()
        if j + 1 < ROWS_PER_W:
            nxt = (j + 1) % 2
            copies[nxt] = pltpu.make_async_copy(
                x_hbm.at[base + j + 1], bufs[nxt], sems[nxt])
            copies[nxt].start()
        row_v = bufs[cur]

        neg = jnp.float32(-jnp.inf)

        def warm_body(i, bs):
            off = i * (4 * L)
            return _fold64(*bs,
                           row_v[pl.ds(off, L)],
                           row_v[pl.ds(off + L, L)],
                           row_v[pl.ds(off + 2 * L, L)],
                           row_v[pl.ds(off + 3 * L, L)])

        w0, w1, w2, w3 = lax.fori_loop(
            0, WARM_CHUNKS // 4, warm_body,
            (jnp.full((L,), neg), jnp.full((L,), neg),
             jnp.full((L,), neg), jnp.full((L,), neg)))
        init = (w0, w1, w2, w3, w3[L - 1])

        def group_body(g, carry):
            b0, b1, b2, b3, thr = carry
            off = g * (GROUP * L)
            cs = [row_v[pl.ds(off + i * L, L)] for i in range(GROUP)]
            pairs = [jnp.maximum(cs[i], cs[i + 1])
                     for i in range(0, GROUP, 2)]
            quads = [jnp.maximum(pairs[i], pairs[i + 1])
                     for i in range(0, len(pairs), 2)]
            gm = quads
            while len(gm) > 1:
                gm = [jnp.maximum(gm[i], gm[i + 1])
                      for i in range(0, len(gm), 2)]

            def do(carry):
                for qi, q in enumerate(quads):
                    def do_quad(carry, qi=qi):
                        cas = [_sort_asc(cs[4 * qi + i]) for i in range(4)]
                        for ca in cas:
                            cmax = ca[L - 1]

                            def m(cr, ca=ca, cmax=cmax):
                                return _merge_sorted(ca, cmax, cr[0], cr[1],
                                                     cr[2], cr[3])

                            carry = lax.cond(cmax > carry[4], m,
                                             lambda cr: cr, carry)
                        return carry

                    carry = lax.cond(_hit(q, carry[4]), do_quad,
                                     lambda cr: cr, carry)
                return carry

            return lax.cond(_hit(gm[0], thr), do, lambda cr: cr, carry)

        b0, b1, b2, b3, _ = lax.fori_loop(WARM_CHUNKS // GROUP, NGROUP,
                                          group_body, init)

        acc = b0 * w_v[j, pl.ds(0, L)]
        acc = acc + b1 * w_v[j, pl.ds(L, L)]
        acc = acc + b2 * w_v[j, pl.ds(2 * L, L)]
        acc = acc + b3 * w_v[j, pl.ds(3 * L, L)]
        s = jnp.sum(acc)
        res = jnp.where(lane == j, s, res)

    res_v[...] = res
    pltpu.sync_copy(res_v, out_hbm.at[wid])


_topk_dot_sc = functools.partial(
    pl.kernel,
    out_type=jax.ShapeDtypeStruct((NWORKERS, L), jnp.float32),
    mesh=plsc.VectorSubcoreMesh(core_axis_name="c", subcore_axis_name="s"),
    scratch_types=[
        pltpu.VMEM((N,), jnp.float32),
        pltpu.VMEM((N,), jnp.float32),
        pltpu.VMEM((ROWS_PER_W, K), jnp.float32),
        pltpu.VMEM((L,), jnp.float32),
        pltpu.SemaphoreType.DMA,
        pltpu.SemaphoreType.DMA,
    ],
    compiler_params=pltpu.CompilerParams(needs_layout_passes=False),
)(_sc_body)


def _logmean_body(x_ref, o_ref):
    x = x_ref[...]
    lane_id = lax.broadcasted_iota(jnp.int32, (NWORKERS, L), 1)
    v = jnp.where(lane_id < ROWS_PER_W, jnp.log(x + TOL), 0.0)
    o_ref[...] = (jnp.sum(v) * (1.0 / B))[None, None]


def kernel(concat_output, sample_loc_prob):
    staged = _topk_dot_sc(concat_output, sample_loc_prob)
    out = pl.pallas_call(
        _logmean_body,
        out_shape=jax.ShapeDtypeStruct((1, 1), jnp.float32),
    )(staged)
    return out[0, 0]
